# baseline (speedup 1.0000x reference)
import jax
import jax.numpy as jnp
from jax.experimental import pallas as pl


_N = 2048


def _nnd_batch_kernel(p1_ref, p2_ref, d1_ref, d2_ref):
    p1 = p1_ref[0]
    p2 = p2_ref[0]
    n1 = jnp.sum(p1 * p1, axis=1, keepdims=True)
    n2 = jnp.sum(p2 * p2, axis=1, keepdims=True)
    ones = jnp.ones_like(n1)
    a = jnp.concatenate([p1, n1, ones], axis=1)
    b = jnp.concatenate([-2.0 * p2, ones, n2], axis=1)
    d = jax.lax.dot_general(
        a, b, (((1,), (1,)), ((), ())),
        preferred_element_type=jnp.float32,
        precision=jax.lax.Precision.HIGHEST,
    )
    d1_ref[0, 0] = jnp.min(d, axis=1)
    d2_ref[0, 0] = jnp.min(d, axis=0)


def kernel(input1, input2):
    bsz, n, _ = input1.shape
    grid = (bsz,)
    out_shape = (
        jax.ShapeDtypeStruct((bsz, 1, n), jnp.float32),
        jax.ShapeDtypeStruct((bsz, 1, n), jnp.float32),
    )
    d1, d2 = pl.pallas_call(
        _nnd_batch_kernel,
        grid=grid,
        in_specs=[
            pl.BlockSpec((1, n, 3), lambda b: (b, 0, 0)),
            pl.BlockSpec((1, n, 3), lambda b: (b, 0, 0)),
        ],
        out_specs=(
            pl.BlockSpec((1, 1, n), lambda b: (b, 0, 0)),
            pl.BlockSpec((1, 1, n), lambda b: (b, 0, 0)),
        ),
        out_shape=out_shape,
    )(input1, input2)
    return d1.reshape(bsz, n), d2.reshape(bsz, n)
